# Initial kernel scaffold; baseline (speedup 1.0000x reference)
import jax
import jax.numpy as jnp
from jax.experimental import pallas as pl


def kernel(x, adj, emb_table, fc_W, fc_b, W1, b1, W2, b2):
    raise NotImplementedError("write your pallas kernel here")


# speedup vs baseline: 1.0242x; 1.0242x over previous
import functools

import jax
import jax.numpy as jnp
from jax.experimental import pallas as pl

N = 10000
BLK = 400


def _prelude_kernel(x_ref, emb_ref, fcw_ref, fcb_ref, w1a_ref, w1b_ref, z1_ref):
    wc = jnp.dot(fcw_ref[:], w1b_ref[:], preferred_element_type=jnp.float32)
    c0 = jnp.dot(fcb_ref[:], w1b_ref[:], preferred_element_type=jnp.float32)
    z1_ref[:] = (
        jnp.dot(x_ref[:], w1a_ref[:], preferred_element_type=jnp.float32)
        + jnp.dot(emb_ref[:], wc, preferred_element_type=jnp.float32)
        + c0
    )


def _pass1_kernel(adj_ref, z1_ref, b1_ref, w2_ref, z2_ref):
    h = jnp.dot(adj_ref[:], z1_ref[:], preferred_element_type=jnp.float32)
    h = jnp.maximum(h + b1_ref[:], 0.0)
    z2_ref[:] = jnp.dot(h, w2_ref[:], preferred_element_type=jnp.float32)


def _pass2_kernel(adj_ref, z2_ref, b2_ref, out_ref):
    o = jnp.dot(adj_ref[:], z2_ref[:], preferred_element_type=jnp.float32)
    o = o + b2_ref[:]
    m = jnp.max(o, axis=1, keepdims=True)
    lse = jnp.log(jnp.sum(jnp.exp(o - m), axis=1, keepdims=True)) + m
    out_ref[:] = o - lse


@functools.partial(jax.jit, static_argnames=())
def kernel(x, adj, emb_table, fc_W, fc_b, W1, b1, W2, b2):
    nfeat = x.shape[1]
    w1a = W1[:nfeat]
    w1b = W1[nfeat:]

    z1 = pl.pallas_call(
        _prelude_kernel,
        out_shape=jax.ShapeDtypeStruct((N, W1.shape[1]), jnp.float32),
    )(x, emb_table, fc_W, fc_b.reshape(1, -1), w1a, w1b)

    grid = (N // BLK,)
    z2 = pl.pallas_call(
        _pass1_kernel,
        grid=grid,
        in_specs=[
            pl.BlockSpec((BLK, N), lambda i: (i, 0)),
            pl.BlockSpec((N, W1.shape[1]), lambda i: (0, 0)),
            pl.BlockSpec((1, b1.shape[0]), lambda i: (0, 0)),
            pl.BlockSpec(W2.shape, lambda i: (0, 0)),
        ],
        out_specs=pl.BlockSpec((BLK, W2.shape[1]), lambda i: (i, 0)),
        out_shape=jax.ShapeDtypeStruct((N, W2.shape[1]), jnp.float32),
    )(adj, z1, b1.reshape(1, -1), W2)

    out = pl.pallas_call(
        _pass2_kernel,
        grid=grid,
        in_specs=[
            pl.BlockSpec((BLK, N), lambda i: (i, 0)),
            pl.BlockSpec((N, W2.shape[1]), lambda i: (0, 0)),
            pl.BlockSpec((1, b2.shape[0]), lambda i: (0, 0)),
        ],
        out_specs=pl.BlockSpec((BLK, W2.shape[1]), lambda i: (i, 0)),
        out_shape=jax.ShapeDtypeStruct((N, W2.shape[1]), jnp.float32),
    )(adj, z2, b2.reshape(1, -1))
    return out
